# Initial kernel scaffold; baseline (speedup 1.0000x reference)
import jax
import jax.numpy as jnp
from jax import lax
from jax.experimental import pallas as pl
from jax.experimental.pallas import tpu as pltpu
from jax.experimental.pallas import tpu_sc as plsc

_L = 16
_R = 128
_N = 32768
_NW = 32
_RPW = _R // _NW
_SEG = _N // _L


def _rev_cumsum_body(x_hbm, out_hbm, xin, xout):
    cid = lax.axis_index("c")
    sid = lax.axis_index("s")
    wid = sid * 2 + cid
    seg_base = lax.iota(jnp.int32, _L) * _SEG

    for k in range(_RPW):
        row = wid * _RPW + k
        pltpu.sync_copy(x_hbm.at[row], xin)

        def p1(i, t):
            idx = jnp.full((_L,), _N - _SEG + i, jnp.int32) - seg_base
            return t + plsc.load_gather(xin, [idx])

        t = lax.fori_loop(0, _SEG, p1, jnp.zeros((_L,), jnp.float32))

        offs = plsc.cumsum(t) - t

        def p3(i, acc):
            gidx = jnp.full((_L,), _N - 1 - i, jnp.int32) - seg_base
            acc = acc + plsc.load_gather(xin, [gidx])
            sidx = jnp.full((_L,), i, jnp.int32) + seg_base
            plsc.store_scatter(xout, [sidx], acc)
            return acc

        lax.fori_loop(0, _SEG, p3, offs)

        pltpu.sync_copy(xout, out_hbm.at[row])


def kernel(x):
    mesh = plsc.VectorSubcoreMesh(core_axis_name="c", subcore_axis_name="s")
    f = pl.kernel(
        _rev_cumsum_body,
        out_type=jax.ShapeDtypeStruct((_R, _N), jnp.float32),
        mesh=mesh,
        scratch_types=[
            pltpu.VMEM((_N,), jnp.float32),
            pltpu.VMEM((_N,), jnp.float32),
        ],
    )
    return f(x)


# speedup vs baseline: 1.1056x; 1.1056x over previous
import jax
import jax.numpy as jnp
from jax import lax
from jax.experimental import pallas as pl
from jax.experimental.pallas import tpu as pltpu
from jax.experimental.pallas import tpu_sc as plsc

_L = 16
_R = 128
_N = 32768
_NW = 32
_RPW = _R // _NW
_SEG = _N // _L


def _rev_cumsum_body(x_hbm, out_hbm, xin, xout):
    cid = lax.axis_index("c")
    sid = lax.axis_index("s")
    wid = sid * 2 + cid
    seg_base = lax.iota(jnp.int32, _L) * _SEG

    for k in range(_RPW):
        row = wid * _RPW + k
        pltpu.sync_copy(x_hbm.at[row], xin)

        def p1(i, t):
            idx = jnp.full((_L,), _N - _SEG + i, jnp.int32) - seg_base
            return t + plsc.load_gather(xin, [idx])

        t = lax.fori_loop(0, _SEG, p1, jnp.zeros((_L,), jnp.float32))

        offs = plsc.cumsum(t) - t

        def p3(i, acc):
            gidx = jnp.full((_L,), _N - 1 - i, jnp.int32) - seg_base
            acc = acc + plsc.load_gather(xin, [gidx])
            sidx = jnp.full((_L,), i, jnp.int32) + seg_base
            plsc.store_scatter(xout, [sidx], acc)
            return acc

        lax.fori_loop(0, _SEG, p3, offs)

        pltpu.sync_copy(xout, out_hbm.at[row])


def kernel(x):
    mesh = plsc.VectorSubcoreMesh(core_axis_name="c", subcore_axis_name="s")
    f = pl.kernel(
        _rev_cumsum_body,
        out_type=jax.ShapeDtypeStruct((_R, _N), jnp.float32),
        mesh=mesh,
        scratch_types=[
            pltpu.VMEM((_N,), jnp.float32),
            pltpu.VMEM((_N,), jnp.float32),
        ],
        compiler_params=pltpu.CompilerParams(needs_layout_passes=False),
    )
    return f(x)


# speedup vs baseline: 1.9525x; 1.7660x over previous
import jax
import jax.numpy as jnp
from jax import lax
from jax.experimental import pallas as pl
from jax.experimental.pallas import tpu as pltpu
from jax.experimental.pallas import tpu_sc as plsc

_L = 16
_R = 128
_N = 32768
_NW = 32
_RPW = _R // _NW
_SEG = _N // _L


def _rev_cumsum_body(x_hbm, out_hbm, xin, xout):
    cid = lax.axis_index("c")
    sid = lax.axis_index("s")
    wid = sid * 2 + cid
    seg_base = lax.iota(jnp.int32, _L) * _SEG

    for k in range(_RPW):
        row = wid * _RPW + k
        pltpu.sync_copy(x_hbm.at[row], xin)

        def p1(i, t):
            idx = jnp.full((_L,), _N - _SEG + i, jnp.int32) - seg_base
            return t + plsc.load_gather(xin, [idx])

        t = plsc.parallel_loop(
            0, _SEG, unroll=8, carry=jnp.zeros((_L,), jnp.float32))(p1)

        offs = plsc.cumsum(t) - t

        def p3(i, acc):
            gidx = jnp.full((_L,), _N - 1 - i, jnp.int32) - seg_base
            acc = acc + plsc.load_gather(xin, [gidx])
            sidx = jnp.full((_L,), i, jnp.int32) + seg_base
            plsc.store_scatter(xout, [sidx], acc)
            return acc

        plsc.parallel_loop(0, _SEG, unroll=8, carry=offs)(p3)

        pltpu.sync_copy(xout, out_hbm.at[row])


def kernel(x):
    mesh = plsc.VectorSubcoreMesh(core_axis_name="c", subcore_axis_name="s")
    f = pl.kernel(
        _rev_cumsum_body,
        out_type=jax.ShapeDtypeStruct((_R, _N), jnp.float32),
        mesh=mesh,
        scratch_types=[
            pltpu.VMEM((_N,), jnp.float32),
            pltpu.VMEM((_N,), jnp.float32),
        ],
        compiler_params=pltpu.CompilerParams(needs_layout_passes=False),
    )
    return f(x)


# speedup vs baseline: 2.0815x; 1.0661x over previous
import jax
import jax.numpy as jnp
from jax import lax
from jax.experimental import pallas as pl
from jax.experimental.pallas import tpu as pltpu
from jax.experimental.pallas import tpu_sc as plsc

_L = 16
_R = 128
_N = 32768
_NW = 32
_RPW = _R // _NW
_CPR = 2
_CH = _N // _CPR
_SEGC = _CH // _L
_NQ = _RPW * _CPR


def _rev_cumsum_body(x_hbm, out_hbm, xin0, xin1, xout0, xout1, sem_in0,
                     sem_in1, sem_out0, sem_out1):
    cid = lax.axis_index("c")
    sid = lax.axis_index("s")
    wid = sid * 2 + cid
    seg_base = lax.iota(jnp.int32, _L) * _SEGC
    xins = (xin0, xin1)
    xouts = (xout0, xout1)
    sems_in = (sem_in0, sem_in1)
    sems_out = (sem_out0, sem_out1)

    def start_in(q):
        r, c = divmod(q, _CPR)
        row = wid * _RPW + r
        col0 = _N - (c + 1) * _CH
        return pltpu.async_copy(
            x_hbm.at[row, pl.ds(col0, _CH)], xins[q % 2], sems_in[q % 2])

    def start_out(q):
        r, c = divmod(q, _CPR)
        row = wid * _RPW + r
        return pltpu.async_copy(
            xouts[q % 2], out_hbm.at[row, pl.ds(c * _CH, _CH)],
            sems_out[q % 2])

    def compute(xin_b, xout_b, carry):
        def p1(i, t):
            idx = jnp.full((_L,), _CH - _SEGC + i, jnp.int32) - seg_base
            return t + plsc.load_gather(xin_b, [idx])

        t = plsc.parallel_loop(
            0, _SEGC, unroll=8, carry=jnp.zeros((_L,), jnp.float32))(p1)

        offs = plsc.cumsum(t) - t + carry
        total = jnp.sum(t)

        def p3(i, acc):
            gidx = jnp.full((_L,), _CH - 1 - i, jnp.int32) - seg_base
            acc = acc + plsc.load_gather(xin_b, [gidx])
            sidx = jnp.full((_L,), i, jnp.int32) + seg_base
            plsc.store_scatter(xout_b, [sidx], acc)
            return acc

        plsc.parallel_loop(0, _SEGC, unroll=8, carry=offs)(p3)
        return carry + total

    h_in = [None] * _NQ
    h_out = [None] * _NQ
    h_in[0] = start_in(0)
    carry = jnp.float32(0.0)
    for q in range(_NQ):
        if q + 1 < _NQ:
            h_in[q + 1] = start_in(q + 1)
        h_in[q].wait()
        if q >= 2:
            h_out[q - 2].wait()
        if q % _CPR == 0:
            carry = jnp.float32(0.0)
        carry = compute(xins[q % 2], xouts[q % 2], carry)
        h_out[q] = start_out(q)
    h_out[_NQ - 2].wait()
    h_out[_NQ - 1].wait()


def kernel(x):
    mesh = plsc.VectorSubcoreMesh(core_axis_name="c", subcore_axis_name="s")
    f = pl.kernel(
        _rev_cumsum_body,
        out_type=jax.ShapeDtypeStruct((_R, _N), jnp.float32),
        mesh=mesh,
        scratch_types=[
            pltpu.VMEM((_CH,), jnp.float32),
            pltpu.VMEM((_CH,), jnp.float32),
            pltpu.VMEM((_CH,), jnp.float32),
            pltpu.VMEM((_CH,), jnp.float32),
            pltpu.SemaphoreType.DMA,
            pltpu.SemaphoreType.DMA,
            pltpu.SemaphoreType.DMA,
            pltpu.SemaphoreType.DMA,
        ],
        compiler_params=pltpu.CompilerParams(needs_layout_passes=False),
    )
    return f(x)
